# baseline (speedup 1.0000x reference)
import functools

import jax
import jax.numpy as jnp
from jax import lax
from jax.experimental import pallas as pl
from jax.experimental.pallas import tpu as pltpu
from jax.experimental.pallas import tpu_sc as plsc

N_NODES = 10000
N_EDGES = 320000
NPAD = 10240
ROWS_PER_TILE = NPAD // 16
EBLK = 64
NBLK_TOTAL = 5120
NBLK32 = NBLK_TOTAL // 32
NBLK16 = NBLK_TOTAL // 16
EPAD = NBLK_TOTAL * EBLK
PB = 16

_MESH = plsc.VectorSubcoreMesh(core_axis_name="c", subcore_axis_name="s")


NBUF = 5


def _agg_piece(x_hbm, src_v, dst_v, bufs, sems, acc_s):
  for b in range(NBUF):
    pltpu.async_copy(x_hbm.at[src_v.at[b]], bufs[b], sems[b])
  for b in range(PB):
    i = b % NBUF
    pltpu.make_async_copy(x_hbm.at[src_v.at[b]], bufs[i], sems[i]).wait()
    pltpu.sync_copy(bufs[i], acc_s.at[dst_v.at[b]], add=True)
    if b + NBUF < PB:
      pltpu.async_copy(x_hbm.at[src_v.at[b + NBUF]], bufs[i], sems[i])


def _make_sc_agg_edge_split(width):
  f32 = jnp.float32
  out_type = jax.ShapeDtypeStruct((2, NPAD, width), f32)
  scratch = [
      pltpu.VMEM((PB, EBLK), jnp.int32),
      pltpu.VMEM((PB, EBLK), jnp.int32),
  ] + [pltpu.VMEM((EBLK, width), f32) for _ in range(NBUF)] + [
      pltpu.VMEM_SHARED((NPAD, width), f32),
  ] + [pltpu.SemaphoreType.DMA for _ in range(NBUF)]

  def body(x_hbm, srcb, dstb, zrow, agg_out, src_v, dst_v, *rest):
    bufs, (acc_s,), sems = rest[:NBUF], rest[NBUF:NBUF + 1], rest[NBUF + 1:]
    c = lax.axis_index("c")
    s = lax.axis_index("s")
    w = c * 16 + s
    r0 = s * ROWS_PER_TILE
    pltpu.sync_copy(zrow, acc_s.at[pl.ds(r0, ROWS_PER_TILE)])
    plsc.subcore_barrier()

    def piece(p, carry):
      blk0 = w * NBLK32 + p * PB
      pltpu.sync_copy(srcb.at[pl.ds(blk0, PB)], src_v)
      pltpu.sync_copy(dstb.at[pl.ds(blk0, PB)], dst_v)
      _agg_piece(x_hbm, src_v, dst_v, bufs, sems, acc_s)
      return carry

    lax.fori_loop(0, NBLK32 // PB, piece, 0)
    plsc.subcore_barrier()
    pltpu.sync_copy(acc_s.at[pl.ds(r0, ROWS_PER_TILE)],
                    agg_out.at[c, pl.ds(r0, ROWS_PER_TILE)])

  return pl.kernel(body, out_type=out_type, mesh=_MESH,
                   scratch_types=scratch)


def _make_sc_deg(dw):
  out_type = jax.ShapeDtypeStruct((2, NPAD, dw), jnp.float32)
  scratch = [
      pltpu.VMEM((PB, EBLK), jnp.int32),
      pltpu.VMEM((EBLK, dw), jnp.float32),
      pltpu.VMEM_SHARED((NPAD, dw), jnp.float32),
  ]

  def body(dstb, zdeg, ones_hbm, deg_out, dst_v, ones_v, acc_s):
    c = lax.axis_index("c")
    s = lax.axis_index("s")
    w = c * 16 + s
    r0 = s * ROWS_PER_TILE
    pltpu.sync_copy(zdeg, acc_s.at[pl.ds(r0, ROWS_PER_TILE)])
    pltpu.sync_copy(ones_hbm, ones_v)
    plsc.subcore_barrier()

    def piece(p, carry):
      pltpu.sync_copy(dstb.at[pl.ds(w * NBLK32 + p * PB, PB)], dst_v)

      def step(b, carry2):
        pltpu.sync_copy(ones_v, acc_s.at[dst_v.at[b]], add=True)
        return carry2

      return lax.fori_loop(0, PB, step, carry)

    lax.fori_loop(0, NBLK32 // PB, piece, 0)
    plsc.subcore_barrier()
    pltpu.sync_copy(acc_s.at[pl.ds(r0, ROWS_PER_TILE)],
                    deg_out.at[c, pl.ds(r0, ROWS_PER_TILE)])

  return pl.kernel(body, out_type=out_type, mesh=_MESH,
                   scratch_types=scratch)


def _make_sc_agg_feat_split():
  out_type = jax.ShapeDtypeStruct((2, NPAD, 128), jnp.float32)
  scratch = [
      pltpu.VMEM((PB, EBLK), jnp.int32),
      pltpu.VMEM((PB, EBLK), jnp.int32),
  ] + [pltpu.VMEM((EBLK, 128), jnp.float32) for _ in range(NBUF)] + [
      pltpu.VMEM_SHARED((NPAD, 128), jnp.float32),
  ] + [pltpu.SemaphoreType.DMA for _ in range(NBUF)]

  def body(x_hbm, srcb, dstb, zrow, agg_out, src_v, dst_v, *rest):
    bufs, (acc_s,), sems = rest[:NBUF], rest[NBUF:NBUF + 1], rest[NBUF + 1:]
    c = lax.axis_index("c")
    s = lax.axis_index("s")
    r0 = s * ROWS_PER_TILE
    pltpu.sync_copy(zrow, acc_s.at[pl.ds(r0, ROWS_PER_TILE)])
    plsc.subcore_barrier()

    def piece(p, carry):
      blk0 = s * NBLK16 + p * PB
      pltpu.sync_copy(srcb.at[c, pl.ds(blk0, PB)], src_v)
      pltpu.sync_copy(dstb.at[pl.ds(blk0, PB)], dst_v)
      _agg_piece(x_hbm, src_v, dst_v, bufs, sems, acc_s)
      return carry

    lax.fori_loop(0, NBLK16 // PB, piece, 0)
    plsc.subcore_barrier()
    pltpu.sync_copy(acc_s.at[pl.ds(r0, ROWS_PER_TILE)],
                    agg_out.at[c, pl.ds(r0, ROWS_PER_TILE)])

  return pl.kernel(body, out_type=out_type, mesh=_MESH,
                   scratch_types=scratch)


_BN = 1000


def _deg_inv(degp_ref):
  deg = degp_ref[0, :, 0:1] + degp_ref[1, :, 0:1]
  return 1.0 / jnp.maximum(deg, 1.0)


def _tc1_body(aggp, degp, x, wl, bl, wr, h1s):
  agg = aggp[0] + aggp[1]
  mean = agg * _deg_inv(degp)
  h = jnp.dot(mean, wl[...], preferred_element_type=jnp.float32)
  h += jnp.dot(x[...], wr[...], preferred_element_type=jnp.float32)
  h = jnp.maximum(h + bl[...], 0.0)
  h1s[0] = h[:, :128]
  h1s[1] = h[:, 128:]


def _tc2_body(agg2, degp, h1s, wl, bl, wr, wl3, wr3, q3l, q3r):
  mean = jnp.concatenate([agg2[0], agg2[1]], axis=1) * _deg_inv(degp)
  h1 = jnp.concatenate([h1s[0], h1s[1]], axis=1)
  h = jnp.dot(mean, wl[...], preferred_element_type=jnp.float32)
  h += jnp.dot(h1, wr[...], preferred_element_type=jnp.float32)
  h2 = jnp.maximum(h + bl[...], 0.0)
  q3l[...] = jnp.dot(h2, wl3[...], preferred_element_type=jnp.float32)
  q3r[...] = jnp.dot(h2, wr3[...], preferred_element_type=jnp.float32)


def _tc3_body(agg3p, degp, q3r, bl, out):
  mean = (agg3p[0, :, :64] + agg3p[1, :, :64]) * _deg_inv(degp)
  z = jnp.maximum(mean + bl[...] + q3r[...], 0.0)
  m = jnp.max(z, axis=-1, keepdims=True)
  e = jnp.exp(z - m)
  out[...] = (z - m) - jnp.log(jnp.sum(e, axis=-1, keepdims=True))


def _rowblk(width):
  return pl.BlockSpec((_BN, width), lambda i: (i, 0))


def _partblk(width):
  return pl.BlockSpec((2, _BN, width), lambda i: (0, i, 0))


def _full2(a, b):
  return pl.BlockSpec((a, b), lambda i: (0, 0))


def kernel(features, edge_index, W_l1, b_l1, W_r1, W_l2, b_l2, W_r2,
           W_l3, b_l3, W_r3):
  f32 = jnp.float32
  src = edge_index[0].astype(jnp.int32)
  dst = edge_index[1].astype(jnp.int32)
  npad_e = EPAD - N_EDGES
  src_p = jnp.concatenate([src, jnp.zeros((npad_e,), jnp.int32)])
  dst_p = jnp.concatenate(
      [dst, N_NODES + (jnp.arange(npad_e, dtype=jnp.int32) % (NPAD - N_NODES))])
  srcb = src_p.reshape(NBLK_TOTAL, EBLK)
  dstb = dst_p.reshape(NBLK_TOTAL, EBLK)
  srcb2 = jnp.stack([srcb, srcb + N_NODES])
  srcb_es = jnp.concatenate(
      [srcb[:NBLK_TOTAL // 2], srcb[NBLK_TOTAL // 2:] + N_NODES])

  zrow128 = jnp.zeros((ROWS_PER_TILE, 128), f32)
  ones128 = jnp.ones((EBLK, 128), f32)

  degp = _make_sc_deg(128)(dstb, zrow128, ones128)

  agg1p = _make_sc_agg_edge_split(128)(
      jnp.concatenate([features, features]), srcb_es, dstb, zrow128)

  h1s = pl.pallas_call(
      _tc1_body,
      grid=(N_NODES // _BN,),
      in_specs=[_partblk(128), _partblk(128), _rowblk(128),
                _full2(128, 256), _full2(1, 256), _full2(128, 256)],
      out_specs=_partblk(128),
      out_shape=jax.ShapeDtypeStruct((2, N_NODES, 128), f32),
  )(agg1p, degp, features, W_l1, b_l1.reshape(1, 256), W_r1)

  agg2 = _make_sc_agg_feat_split()(
      h1s.reshape(2 * N_NODES, 128), srcb2, dstb, zrow128)

  q3l, q3r = pl.pallas_call(
      _tc2_body,
      grid=(N_NODES // _BN,),
      in_specs=[_partblk(128), _partblk(128), _partblk(128),
                _full2(256, 256), _full2(1, 256), _full2(256, 256),
                _full2(256, 64), _full2(256, 64)],
      out_specs=[_rowblk(64), _rowblk(64)],
      out_shape=[jax.ShapeDtypeStruct((N_NODES, 64), f32),
                 jax.ShapeDtypeStruct((N_NODES, 64), f32)],
  )(agg2, degp, h1s, W_l2, b_l2.reshape(1, 256), W_r2, W_l3, W_r3)

  q3 = jnp.concatenate([q3l, q3r], axis=1)
  agg3p = _make_sc_agg_edge_split(128)(
      jnp.concatenate([q3, q3]), srcb_es, dstb, zrow128)

  out = pl.pallas_call(
      _tc3_body,
      grid=(N_NODES // _BN,),
      in_specs=[_partblk(128), _partblk(128), _rowblk(64),
                _full2(1, 64)],
      out_specs=_rowblk(64),
      out_shape=jax.ShapeDtypeStruct((N_NODES, 64), f32),
  )(agg3p, degp, q3r, b_l3.reshape(1, 64))
  return out


# speedup vs baseline: 1.1466x; 1.1466x over previous
import functools

import jax
import jax.numpy as jnp
from jax import lax
from jax.experimental import pallas as pl
from jax.experimental.pallas import tpu as pltpu
from jax.experimental.pallas import tpu_sc as plsc

N_NODES = 10000
N_EDGES = 320000
NPAD = 10240
ROWS_PER_TILE = NPAD // 16
EBLK = 128
NBLK_TOTAL = 2560
NBLK32 = NBLK_TOTAL // 32
NBLK16 = NBLK_TOTAL // 16
EPAD = NBLK_TOTAL * EBLK
PB = 40

_MESH = plsc.VectorSubcoreMesh(core_axis_name="c", subcore_axis_name="s")


NBUF = 2


def _agg_piece(x_hbm, src_v, dst_v, bufs, sems, acc_s):
  for b in range(NBUF):
    pltpu.async_copy(x_hbm.at[src_v.at[b]], bufs[b], sems[b])
  for b in range(PB):
    i = b % NBUF
    pltpu.make_async_copy(x_hbm.at[src_v.at[b]], bufs[i], sems[i]).wait()
    pltpu.sync_copy(bufs[i], acc_s.at[dst_v.at[b]], add=True)
    if b + NBUF < PB:
      pltpu.async_copy(x_hbm.at[src_v.at[b + NBUF]], bufs[i], sems[i])


def _make_sc_agg_edge_split(width):
  f32 = jnp.float32
  out_type = jax.ShapeDtypeStruct((2, NPAD, width), f32)
  scratch = [
      pltpu.VMEM((PB, EBLK), jnp.int32),
      pltpu.VMEM((PB, EBLK), jnp.int32),
  ] + [pltpu.VMEM((EBLK, width), f32) for _ in range(NBUF)] + [
      pltpu.VMEM_SHARED((NPAD, width), f32),
  ] + [pltpu.SemaphoreType.DMA for _ in range(NBUF)]

  def body(x_hbm, srcb, dstb, zrow, agg_out, src_v, dst_v, *rest):
    bufs, (acc_s,), sems = rest[:NBUF], rest[NBUF:NBUF + 1], rest[NBUF + 1:]
    c = lax.axis_index("c")
    s = lax.axis_index("s")
    w = c * 16 + s
    r0 = s * ROWS_PER_TILE
    pltpu.sync_copy(zrow, acc_s.at[pl.ds(r0, ROWS_PER_TILE)])
    plsc.subcore_barrier()

    def piece(p, carry):
      blk0 = w * NBLK32 + p * PB
      pltpu.sync_copy(srcb.at[pl.ds(blk0, PB)], src_v)
      pltpu.sync_copy(dstb.at[pl.ds(blk0, PB)], dst_v)
      _agg_piece(x_hbm, src_v, dst_v, bufs, sems, acc_s)
      return carry

    lax.fori_loop(0, NBLK32 // PB, piece, 0)
    plsc.subcore_barrier()
    pltpu.sync_copy(acc_s.at[pl.ds(r0, ROWS_PER_TILE)],
                    agg_out.at[c, pl.ds(r0, ROWS_PER_TILE)])

  return pl.kernel(body, out_type=out_type, mesh=_MESH,
                   scratch_types=scratch)


def _make_sc_deg(dw):
  out_type = jax.ShapeDtypeStruct((2, NPAD, dw), jnp.float32)
  scratch = [
      pltpu.VMEM((PB, EBLK), jnp.int32),
      pltpu.VMEM((EBLK, dw), jnp.float32),
      pltpu.VMEM_SHARED((NPAD, dw), jnp.float32),
  ]

  def body(dstb, zdeg, ones_hbm, deg_out, dst_v, ones_v, acc_s):
    c = lax.axis_index("c")
    s = lax.axis_index("s")
    w = c * 16 + s
    r0 = s * ROWS_PER_TILE
    pltpu.sync_copy(zdeg, acc_s.at[pl.ds(r0, ROWS_PER_TILE)])
    pltpu.sync_copy(ones_hbm, ones_v)
    plsc.subcore_barrier()

    def piece(p, carry):
      pltpu.sync_copy(dstb.at[pl.ds(w * NBLK32 + p * PB, PB)], dst_v)

      def step(b, carry2):
        pltpu.sync_copy(ones_v, acc_s.at[dst_v.at[b]], add=True)
        return carry2

      return lax.fori_loop(0, PB, step, carry)

    lax.fori_loop(0, NBLK32 // PB, piece, 0)
    plsc.subcore_barrier()
    pltpu.sync_copy(acc_s.at[pl.ds(r0, ROWS_PER_TILE)],
                    deg_out.at[c, pl.ds(r0, ROWS_PER_TILE)])

  return pl.kernel(body, out_type=out_type, mesh=_MESH,
                   scratch_types=scratch)


def _make_sc_agg_feat_split():
  out_type = jax.ShapeDtypeStruct((2, NPAD, 128), jnp.float32)
  scratch = [
      pltpu.VMEM((PB, EBLK), jnp.int32),
      pltpu.VMEM((PB, EBLK), jnp.int32),
  ] + [pltpu.VMEM((EBLK, 128), jnp.float32) for _ in range(NBUF)] + [
      pltpu.VMEM_SHARED((NPAD, 128), jnp.float32),
  ] + [pltpu.SemaphoreType.DMA for _ in range(NBUF)]

  def body(x_hbm, srcb, dstb, zrow, agg_out, src_v, dst_v, *rest):
    bufs, (acc_s,), sems = rest[:NBUF], rest[NBUF:NBUF + 1], rest[NBUF + 1:]
    c = lax.axis_index("c")
    s = lax.axis_index("s")
    r0 = s * ROWS_PER_TILE
    pltpu.sync_copy(zrow, acc_s.at[pl.ds(r0, ROWS_PER_TILE)])
    plsc.subcore_barrier()

    def piece(p, carry):
      blk0 = s * NBLK16 + p * PB
      pltpu.sync_copy(srcb.at[c, pl.ds(blk0, PB)], src_v)
      pltpu.sync_copy(dstb.at[pl.ds(blk0, PB)], dst_v)
      _agg_piece(x_hbm, src_v, dst_v, bufs, sems, acc_s)
      return carry

    lax.fori_loop(0, NBLK16 // PB, piece, 0)
    plsc.subcore_barrier()
    pltpu.sync_copy(acc_s.at[pl.ds(r0, ROWS_PER_TILE)],
                    agg_out.at[c, pl.ds(r0, ROWS_PER_TILE)])

  return pl.kernel(body, out_type=out_type, mesh=_MESH,
                   scratch_types=scratch)


_BN = 1000


def _deg_inv(degp_ref):
  deg = degp_ref[0, :, 0:1] + degp_ref[1, :, 0:1]
  return 1.0 / jnp.maximum(deg, 1.0)


def _tc1_body(aggp, degp, x, wl, bl, wr, h1s):
  agg = aggp[0] + aggp[1]
  mean = agg * _deg_inv(degp)
  h = jnp.dot(mean, wl[...], preferred_element_type=jnp.float32)
  h += jnp.dot(x[...], wr[...], preferred_element_type=jnp.float32)
  h = jnp.maximum(h + bl[...], 0.0)
  h1s[0] = h[:, :128]
  h1s[1] = h[:, 128:]


def _tc2_body(agg2, degp, h1s, wl, bl, wr, wl3, wr3, q3l, q3r):
  mean = jnp.concatenate([agg2[0], agg2[1]], axis=1) * _deg_inv(degp)
  h1 = jnp.concatenate([h1s[0], h1s[1]], axis=1)
  h = jnp.dot(mean, wl[...], preferred_element_type=jnp.float32)
  h += jnp.dot(h1, wr[...], preferred_element_type=jnp.float32)
  h2 = jnp.maximum(h + bl[...], 0.0)
  q3l[...] = jnp.dot(h2, wl3[...], preferred_element_type=jnp.float32)
  q3r[...] = jnp.dot(h2, wr3[...], preferred_element_type=jnp.float32)


def _tc3_body(agg3p, degp, q3r, bl, out):
  mean = (agg3p[0, :, :64] + agg3p[1, :, :64]) * _deg_inv(degp)
  z = jnp.maximum(mean + bl[...] + q3r[...], 0.0)
  m = jnp.max(z, axis=-1, keepdims=True)
  e = jnp.exp(z - m)
  out[...] = (z - m) - jnp.log(jnp.sum(e, axis=-1, keepdims=True))


def _rowblk(width):
  return pl.BlockSpec((_BN, width), lambda i: (i, 0))


def _partblk(width):
  return pl.BlockSpec((2, _BN, width), lambda i: (0, i, 0))


def _full2(a, b):
  return pl.BlockSpec((a, b), lambda i: (0, 0))


def kernel(features, edge_index, W_l1, b_l1, W_r1, W_l2, b_l2, W_r2,
           W_l3, b_l3, W_r3):
  f32 = jnp.float32
  src = edge_index[0].astype(jnp.int32)
  dst = edge_index[1].astype(jnp.int32)
  npad_e = EPAD - N_EDGES
  src_p = jnp.concatenate([src, jnp.zeros((npad_e,), jnp.int32)])
  dst_p = jnp.concatenate(
      [dst, N_NODES + (jnp.arange(npad_e, dtype=jnp.int32) % (NPAD - N_NODES))])
  srcb = src_p.reshape(NBLK_TOTAL, EBLK)
  dstb = dst_p.reshape(NBLK_TOTAL, EBLK)
  srcb2 = jnp.stack([srcb, srcb + N_NODES])
  srcb_es = jnp.concatenate(
      [srcb[:NBLK_TOTAL // 2], srcb[NBLK_TOTAL // 2:] + N_NODES])

  zrow128 = jnp.zeros((ROWS_PER_TILE, 128), f32)
  ones128 = jnp.ones((EBLK, 128), f32)

  degp = _make_sc_deg(128)(dstb, zrow128, ones128)

  agg1p = _make_sc_agg_edge_split(128)(
      jnp.concatenate([features, features]), srcb_es, dstb, zrow128)

  h1s = pl.pallas_call(
      _tc1_body,
      grid=(N_NODES // _BN,),
      in_specs=[_partblk(128), _partblk(128), _rowblk(128),
                _full2(128, 256), _full2(1, 256), _full2(128, 256)],
      out_specs=_partblk(128),
      out_shape=jax.ShapeDtypeStruct((2, N_NODES, 128), f32),
  )(agg1p, degp, features, W_l1, b_l1.reshape(1, 256), W_r1)

  agg2 = _make_sc_agg_feat_split()(
      h1s.reshape(2 * N_NODES, 128), srcb2, dstb, zrow128)

  q3l, q3r = pl.pallas_call(
      _tc2_body,
      grid=(N_NODES // _BN,),
      in_specs=[_partblk(128), _partblk(128), _partblk(128),
                _full2(256, 256), _full2(1, 256), _full2(256, 256),
                _full2(256, 64), _full2(256, 64)],
      out_specs=[_rowblk(64), _rowblk(64)],
      out_shape=[jax.ShapeDtypeStruct((N_NODES, 64), f32),
                 jax.ShapeDtypeStruct((N_NODES, 64), f32)],
  )(agg2, degp, h1s, W_l2, b_l2.reshape(1, 256), W_r2, W_l3, W_r3)

  q3 = jnp.concatenate([q3l, q3r], axis=1)
  agg3p = _make_sc_agg_edge_split(128)(
      jnp.concatenate([q3, q3]), srcb_es, dstb, zrow128)

  out = pl.pallas_call(
      _tc3_body,
      grid=(N_NODES // _BN,),
      in_specs=[_partblk(128), _partblk(128), _rowblk(64),
                _full2(1, 64)],
      out_specs=_rowblk(64),
      out_shape=jax.ShapeDtypeStruct((N_NODES, 64), f32),
  )(agg3p, degp, q3r, b_l3.reshape(1, 64))
  return out


# speedup vs baseline: 1.1475x; 1.0008x over previous
import jax
import jax.numpy as jnp
from jax import lax
from jax.experimental import pallas as pl
from jax.experimental.pallas import tpu as pltpu
from jax.experimental.pallas import tpu_sc as plsc

N_NODES = 10000
N_EDGES = 320000
NPAD = 10240
ROWS_PER_TILE = NPAD // 16
EBLK = 128
NBLK_TOTAL = 2560
NBLK32 = NBLK_TOTAL // 32
NBLK16 = NBLK_TOTAL // 16
EPAD = NBLK_TOTAL * EBLK
PB = 40

_MESH = plsc.VectorSubcoreMesh(core_axis_name="c", subcore_axis_name="s")


NBUF = 2


def _agg_piece(x_hbm, src_v, dst_v, bufs, sems, acc_s):
  for b in range(NBUF):
    pltpu.async_copy(x_hbm.at[src_v.at[b]], bufs[b], sems[b])
  for b in range(PB):
    i = b % NBUF
    pltpu.make_async_copy(x_hbm.at[src_v.at[b]], bufs[i], sems[i]).wait()
    pltpu.sync_copy(bufs[i], acc_s.at[dst_v.at[b]], add=True)
    if b + NBUF < PB:
      pltpu.async_copy(x_hbm.at[src_v.at[b + NBUF]], bufs[i], sems[i])


def _make_sc_agg_edge_split(width):
  f32 = jnp.float32
  out_type = jax.ShapeDtypeStruct((2, NPAD, width), f32)
  scratch = [
      pltpu.VMEM((PB, EBLK), jnp.int32),
      pltpu.VMEM((PB, EBLK), jnp.int32),
  ] + [pltpu.VMEM((EBLK, width), f32) for _ in range(NBUF)] + [
      pltpu.VMEM_SHARED((NPAD, width), f32),
  ] + [pltpu.SemaphoreType.DMA for _ in range(NBUF)]

  def body(x_hbm, srcb, dstb, zrow, agg_out, src_v, dst_v, *rest):
    bufs, (acc_s,), sems = rest[:NBUF], rest[NBUF:NBUF + 1], rest[NBUF + 1:]
    c = lax.axis_index("c")
    s = lax.axis_index("s")
    w = c * 16 + s
    r0 = s * ROWS_PER_TILE
    pltpu.sync_copy(zrow, acc_s.at[pl.ds(r0, ROWS_PER_TILE)])
    plsc.subcore_barrier()

    def piece(p, carry):
      blk0 = w * NBLK32 + p * PB
      pltpu.sync_copy(srcb.at[pl.ds(blk0, PB)], src_v)
      pltpu.sync_copy(dstb.at[pl.ds(blk0, PB)], dst_v)
      _agg_piece(x_hbm, src_v, dst_v, bufs, sems, acc_s)
      return carry

    lax.fori_loop(0, NBLK32 // PB, piece, 0)
    plsc.subcore_barrier()
    pltpu.sync_copy(acc_s.at[pl.ds(r0, ROWS_PER_TILE)],
                    agg_out.at[c, pl.ds(r0, ROWS_PER_TILE)])

  return pl.kernel(body, out_type=out_type, mesh=_MESH,
                   scratch_types=scratch)


def _make_sc_deg(dw):
  out_type = jax.ShapeDtypeStruct((2, NPAD, dw), jnp.float32)
  scratch = [
      pltpu.VMEM((PB, EBLK), jnp.int32),
      pltpu.VMEM((EBLK, dw), jnp.float32),
      pltpu.VMEM_SHARED((NPAD, dw), jnp.float32),
  ]

  def body(dstb, zdeg, ones_hbm, deg_out, dst_v, ones_v, acc_s):
    c = lax.axis_index("c")
    s = lax.axis_index("s")
    w = c * 16 + s
    r0 = s * ROWS_PER_TILE
    pltpu.sync_copy(zdeg, acc_s.at[pl.ds(r0, ROWS_PER_TILE)])
    pltpu.sync_copy(ones_hbm, ones_v)
    plsc.subcore_barrier()

    def piece(p, carry):
      pltpu.sync_copy(dstb.at[pl.ds(w * NBLK32 + p * PB, PB)], dst_v)

      def step(b, carry2):
        pltpu.sync_copy(ones_v, acc_s.at[dst_v.at[b]], add=True)
        return carry2

      return lax.fori_loop(0, PB, step, carry)

    lax.fori_loop(0, NBLK32 // PB, piece, 0)
    plsc.subcore_barrier()
    pltpu.sync_copy(acc_s.at[pl.ds(r0, ROWS_PER_TILE)],
                    deg_out.at[c, pl.ds(r0, ROWS_PER_TILE)])

  return pl.kernel(body, out_type=out_type, mesh=_MESH,
                   scratch_types=scratch)


def _make_sc_agg_feat_split():
  out_type = jax.ShapeDtypeStruct((2, NPAD, 128), jnp.float32)
  scratch = [
      pltpu.VMEM((PB, EBLK), jnp.int32),
      pltpu.VMEM((PB, EBLK), jnp.int32),
  ] + [pltpu.VMEM((EBLK, 128), jnp.float32) for _ in range(NBUF)] + [
      pltpu.VMEM_SHARED((NPAD, 128), jnp.float32),
  ] + [pltpu.SemaphoreType.DMA for _ in range(NBUF)]

  def body(x_hbm, srcb, dstb, zrow, agg_out, src_v, dst_v, *rest):
    bufs, (acc_s,), sems = rest[:NBUF], rest[NBUF:NBUF + 1], rest[NBUF + 1:]
    c = lax.axis_index("c")
    s = lax.axis_index("s")
    r0 = s * ROWS_PER_TILE
    pltpu.sync_copy(zrow, acc_s.at[pl.ds(r0, ROWS_PER_TILE)])
    plsc.subcore_barrier()

    def piece(p, carry):
      blk0 = s * NBLK16 + p * PB
      pltpu.sync_copy(srcb.at[c, pl.ds(blk0, PB)], src_v)
      pltpu.sync_copy(dstb.at[pl.ds(blk0, PB)], dst_v)
      _agg_piece(x_hbm, src_v, dst_v, bufs, sems, acc_s)
      return carry

    lax.fori_loop(0, NBLK16 // PB, piece, 0)
    plsc.subcore_barrier()
    pltpu.sync_copy(acc_s.at[pl.ds(r0, ROWS_PER_TILE)],
                    agg_out.at[c, pl.ds(r0, ROWS_PER_TILE)])

  return pl.kernel(body, out_type=out_type, mesh=_MESH,
                   scratch_types=scratch)


_BN = 1000


def _deg_inv(degp_ref):
  deg = degp_ref[0, :, 0:1] + degp_ref[1, :, 0:1]
  return 1.0 / jnp.maximum(deg, 1.0)


def _tc1_body(aggp, degp, x, wl, bl, wr, h1s):
  agg = aggp[0] + aggp[1]
  mean = agg * _deg_inv(degp)
  h = jnp.dot(mean, wl[...], preferred_element_type=jnp.float32)
  h += jnp.dot(x[...], wr[...], preferred_element_type=jnp.float32)
  h = jnp.maximum(h + bl[...], 0.0)
  h1s[0] = h[:, :128]
  h1s[1] = h[:, 128:]


def _tc2_body(agg2, degp, h1s, wl, bl, wr, wl3, wr3, q3l, q3r):
  mean = jnp.concatenate([agg2[0], agg2[1]], axis=1) * _deg_inv(degp)
  h1 = jnp.concatenate([h1s[0], h1s[1]], axis=1)
  h = jnp.dot(mean, wl[...], preferred_element_type=jnp.float32)
  h += jnp.dot(h1, wr[...], preferred_element_type=jnp.float32)
  h2 = jnp.maximum(h + bl[...], 0.0)
  q3l[...] = jnp.dot(h2, wl3[...], preferred_element_type=jnp.float32)
  q3r[...] = jnp.dot(h2, wr3[...], preferred_element_type=jnp.float32)


def _tc3_body(agg3p, degp, q3r, bl, out):
  mean = (agg3p[0, :, :64] + agg3p[1, :, :64]) * _deg_inv(degp)
  z = jnp.maximum(mean + bl[...] + q3r[...], 0.0)
  m = jnp.max(z, axis=-1, keepdims=True)
  e = jnp.exp(z - m)
  out[...] = (z - m) - jnp.log(jnp.sum(e, axis=-1, keepdims=True))


def _rowblk(width):
  return pl.BlockSpec((_BN, width), lambda i: (i, 0))


def _partblk(width):
  return pl.BlockSpec((2, _BN, width), lambda i: (0, i, 0))


def _full2(a, b):
  return pl.BlockSpec((a, b), lambda i: (0, 0))


def kernel(features, edge_index, W_l1, b_l1, W_r1, W_l2, b_l2, W_r2,
           W_l3, b_l3, W_r3):
  f32 = jnp.float32
  src = edge_index[0].astype(jnp.int32)
  dst = edge_index[1].astype(jnp.int32)
  npad_e = EPAD - N_EDGES
  src_p = jnp.concatenate([src, jnp.zeros((npad_e,), jnp.int32)])
  dst_p = jnp.concatenate(
      [dst, N_NODES + (jnp.arange(npad_e, dtype=jnp.int32) % (NPAD - N_NODES))])
  srcb = src_p.reshape(NBLK_TOTAL, EBLK)
  dstb = dst_p.reshape(NBLK_TOTAL, EBLK)
  srcb2 = jnp.stack([srcb, srcb + N_NODES])
  srcb_es = jnp.concatenate(
      [srcb[:NBLK_TOTAL // 2], srcb[NBLK_TOTAL // 2:] + N_NODES])

  zrow128 = jnp.zeros((ROWS_PER_TILE, 128), f32)
  ones128 = jnp.ones((EBLK, 128), f32)

  degp = _make_sc_deg(128)(dstb, zrow128, ones128)

  agg1p = _make_sc_agg_edge_split(128)(
      jnp.concatenate([features, features]), srcb_es, dstb, zrow128)

  h1s = pl.pallas_call(
      _tc1_body,
      grid=(N_NODES // _BN,),
      in_specs=[_partblk(128), _partblk(128), _rowblk(128),
                _full2(128, 256), _full2(1, 256), _full2(128, 256)],
      out_specs=_partblk(128),
      out_shape=jax.ShapeDtypeStruct((2, N_NODES, 128), f32),
  )(agg1p, degp, features, W_l1, b_l1.reshape(1, 256), W_r1)

  agg2 = _make_sc_agg_feat_split()(
      h1s.reshape(2 * N_NODES, 128), srcb2, dstb, zrow128)

  q3l, q3r = pl.pallas_call(
      _tc2_body,
      grid=(N_NODES // _BN,),
      in_specs=[_partblk(128), _partblk(128), _partblk(128),
                _full2(256, 256), _full2(1, 256), _full2(256, 256),
                _full2(256, 64), _full2(256, 64)],
      out_specs=[_rowblk(64), _rowblk(64)],
      out_shape=[jax.ShapeDtypeStruct((N_NODES, 64), f32),
                 jax.ShapeDtypeStruct((N_NODES, 64), f32)],
  )(agg2, degp, h1s, W_l2, b_l2.reshape(1, 256), W_r2, W_l3, W_r3)

  q3 = jnp.concatenate([q3l, q3r], axis=1)
  agg3p = _make_sc_agg_edge_split(128)(
      jnp.concatenate([q3, q3]), srcb_es, dstb, zrow128)

  out = pl.pallas_call(
      _tc3_body,
      grid=(N_NODES // _BN,),
      in_specs=[_partblk(128), _partblk(128), _rowblk(64),
                _full2(1, 64)],
      out_specs=_rowblk(64),
      out_shape=jax.ShapeDtypeStruct((N_NODES, 64), f32),
  )(agg3p, degp, q3r, b_l3.reshape(1, 64))
  return out
